# Initial kernel scaffold; baseline (speedup 1.0000x reference)
import jax
import jax.numpy as jnp
from jax.experimental import pallas as pl


def kernel(x):
    raise NotImplementedError("write your pallas kernel here")


# speedup vs baseline: 8.0084x; 8.0084x over previous
import functools

import jax
import jax.numpy as jnp
from jax.experimental import pallas as pl
from jax.experimental.pallas import tpu as pltpu

TOP_K = 122
EPS = 1e-06
import numpy as np

INT_MIN = np.int32(-2147483648)


def _toprank_gem_kernel(x_ref, o_ref, *, k):
    v = x_ref[...]
    b = jax.lax.bitcast_convert_type(v, jnp.int32)
    s = jnp.where(b >= 0, b, jnp.bitwise_not(b) ^ INT_MIN)

    rows = v.shape[0]
    prefix_u = jnp.zeros((rows, 1), jnp.int32)
    for bit in range(31, -1, -1):
        bit_val = np.int32(-2147483648) if bit == 31 else np.int32(1 << bit)
        cand_u = prefix_u | bit_val
        cand_s = cand_u ^ INT_MIN
        cnt = jnp.sum((s >= cand_s).astype(jnp.int32), axis=1, keepdims=True)
        prefix_u = jnp.where(cnt >= k, cand_u, prefix_u)

    t_s = prefix_u ^ INT_MIN
    t_bits = jnp.where(prefix_u < 0, t_s, jnp.bitwise_not(prefix_u))
    t_f = jax.lax.bitcast_convert_type(t_bits, jnp.float32)

    gt = s > t_s
    cnt_gt = jnp.sum(gt.astype(jnp.int32), axis=1, keepdims=True)
    vc = jnp.maximum(v, EPS)
    f = vc * vc * vc
    sum_gt = jnp.sum(jnp.where(gt, f, 0.0), axis=1, keepdims=True)

    tc = jnp.maximum(t_f, EPS)
    ft = tc * tc * tc
    total = sum_gt + (k - cnt_gt).astype(jnp.float32) * ft
    pooled = total * (1.0 / k)
    o_ref[...] = jnp.exp(jnp.log(pooled) * (1.0 / 3.0))


@jax.jit
def kernel(x):
    N, C, H, W = x.shape
    L = H * W
    k = TOP_K
    xf = x.reshape(N * C, L)
    rows = N * C
    R = 256
    grid = (rows // R,)
    out = pl.pallas_call(
        functools.partial(_toprank_gem_kernel, k=k),
        grid=grid,
        in_specs=[pl.BlockSpec((R, L), lambda i: (i, 0))],
        out_specs=pl.BlockSpec((R, 1), lambda i: (i, 0)),
        out_shape=jax.ShapeDtypeStruct((rows, 1), jnp.float32),
        compiler_params=pltpu.CompilerParams(
            dimension_semantics=("parallel",),
        ),
    )(xf)
    return out.reshape(N, C, 1)


# speedup vs baseline: 9.2511x; 1.1552x over previous
import functools

import jax
import jax.numpy as jnp
from jax.experimental import pallas as pl
from jax.experimental.pallas import tpu as pltpu

TOP_K = 122
EPS = 1e-06
import numpy as np

INT_MIN = np.int32(-2147483648)


def _toprank_gem_kernel(x_ref, o_ref, *, k):
    R = x_ref.shape[0]
    v = x_ref[...].reshape(R, -1)
    b = jax.lax.bitcast_convert_type(v, jnp.int32)
    s = jnp.where(b >= 0, b, jnp.bitwise_not(b) ^ INT_MIN)

    rows = v.shape[0]
    prefix_u = jnp.zeros((rows, 1), jnp.int32)
    for bit in range(31, -1, -1):
        bit_val = np.int32(-2147483648) if bit == 31 else np.int32(1 << bit)
        cand_u = prefix_u | bit_val
        cand_s = cand_u ^ INT_MIN
        cnt = jnp.sum((s >= cand_s).astype(jnp.int32), axis=1, keepdims=True)
        prefix_u = jnp.where(cnt >= k, cand_u, prefix_u)

    t_s = prefix_u ^ INT_MIN
    t_bits = jnp.where(prefix_u < 0, t_s, jnp.bitwise_not(prefix_u))
    t_f = jax.lax.bitcast_convert_type(t_bits, jnp.float32)

    gt = s > t_s
    cnt_gt = jnp.sum(gt.astype(jnp.int32), axis=1, keepdims=True)
    vc = jnp.maximum(v, EPS)
    f = vc * vc * vc
    sum_gt = jnp.sum(jnp.where(gt, f, 0.0), axis=1, keepdims=True)

    tc = jnp.maximum(t_f, EPS)
    ft = tc * tc * tc
    total = sum_gt + (k - cnt_gt).astype(jnp.float32) * ft
    pooled = total * (1.0 / k)
    o_ref[...] = jnp.exp(jnp.log(pooled) * (1.0 / 3.0))


@jax.jit
def kernel(x):
    N, C, H, W = x.shape
    L = H * W
    k = TOP_K
    xf = x.reshape(N * C, H, W)
    rows = N * C
    R = 256
    grid = (rows // R,)
    out = pl.pallas_call(
        functools.partial(_toprank_gem_kernel, k=k),
        grid=grid,
        in_specs=[pl.BlockSpec((R, H, W), lambda i: (i, 0, 0))],
        out_specs=pl.BlockSpec((R, 1), lambda i: (i, 0)),
        out_shape=jax.ShapeDtypeStruct((rows, 1), jnp.float32),
        compiler_params=pltpu.CompilerParams(
            dimension_semantics=("parallel",),
        ),
    )(xf)
    return out.reshape(N, C, 1)
